# Initial kernel scaffold; baseline (speedup 1.0000x reference)
import jax
import jax.numpy as jnp
from jax.experimental import pallas as pl


def kernel(state, action_table):
    raise NotImplementedError("write your pallas kernel here")


# speedup vs baseline: 4.2353x; 4.2353x over previous
import functools

import jax
import jax.numpy as jnp
from jax.experimental import pallas as pl
from jax.experimental.pallas import tpu as pltpu

NUM_METRIC = 10
NUM_TASK = 8
NUM_ACTIONS = 1024
ROWS_PER_BLOCK = 1024


def _onehot_kernel(x_ref, table_ref, probs_ref, logits_ref):
    x = x_ref[...]
    metric = x[:, 1:1 + NUM_METRIC]
    task = x[:, 1 + NUM_METRIC:1 + NUM_METRIC + NUM_TASK]
    metric_idx = jnp.argmax(metric, axis=-1)
    task_idx = jnp.argmax(task, axis=-1)

    table = table_ref[...].astype(jnp.float32)
    task_oh = (jax.lax.broadcasted_iota(jnp.int32, task.shape, 1)
               == task_idx[:, None]).astype(jnp.float32)
    metric_oh = (jax.lax.broadcasted_iota(jnp.int32, metric.shape, 1)
                 == metric_idx[:, None]).astype(jnp.float32)
    rows = jax.lax.dot_general(
        task_oh, table,
        dimension_numbers=(((1,), (0,)), ((), ())),
        preferred_element_type=jnp.float32)
    action = jnp.sum(rows * metric_oh, axis=-1, keepdims=True)

    col = jax.lax.broadcasted_iota(jnp.int32, probs_ref.shape, 1)
    probs = (col == action.astype(jnp.int32)).astype(jnp.float32)
    probs_ref[...] = probs
    logits_ref[...] = probs * 1000000.0 - 1000000.0


@jax.jit
def kernel(state, action_table):
    B = state.shape[0]
    grid = (B // ROWS_PER_BLOCK,)
    probs, logits = pl.pallas_call(
        _onehot_kernel,
        grid=grid,
        in_specs=[
            pl.BlockSpec((ROWS_PER_BLOCK, 128), lambda i: (i, 0)),
            pl.BlockSpec((NUM_TASK, NUM_METRIC), lambda i: (0, 0)),
        ],
        out_specs=[
            pl.BlockSpec((ROWS_PER_BLOCK, NUM_ACTIONS), lambda i: (i, 0)),
            pl.BlockSpec((ROWS_PER_BLOCK, NUM_ACTIONS), lambda i: (i, 0)),
        ],
        out_shape=[
            jax.ShapeDtypeStruct((B, NUM_ACTIONS), jnp.float32),
            jax.ShapeDtypeStruct((B, NUM_ACTIONS), jnp.float32),
        ],
        compiler_params=pltpu.CompilerParams(
            dimension_semantics=("arbitrary",)),
    )(state, action_table)
    feature_vector = jnp.zeros((B, 1), dtype=jnp.float32)
    return (probs, logits, probs, feature_vector)
